# baseline (speedup 1.0000x reference)
import functools

import jax
import jax.numpy as jnp
from jax import lax
from jax.experimental import pallas as pl
from jax.experimental.pallas import tpu as pltpu
from jax.experimental.pallas import tpu_sc as plsc

_BIG = 3.0e38
_NC = 2
_NS = 16


def _proj_body(x_ref, Ws_ref, bs_ref, Wf_ref, bf_ref, co_ref, fe_ref):
    xa = x_ref[0]
    co_ref[0] = jnp.dot(xa, Ws_ref[...],
                        preferred_element_type=jnp.float32) + bs_ref[...]
    fe_ref[0] = jnp.dot(xa, Wf_ref[...],
                        preferred_element_type=jnp.float32) + bf_ref[...]


def _final_body(x_ref, col_ref, Wo_ref, bo_ref, out_ref, *, F):
    Wo = Wo_ref[...]
    acc = jnp.dot(x_ref[...], Wo[:F], preferred_element_type=jnp.float32)
    acc += jnp.dot(col_ref[...], Wo[F:], preferred_element_type=jnp.float32)
    out_ref[...] = jnp.tanh(acc + bo_ref[...])


def _sort16(k, v):
    return plsc.sort_key_val(k, v)


def _round_bf16(x):
    bits = lax.bitcast_convert_type(x, jnp.int32)
    lsb = lax.shift_right_logical(bits, 16) & 1
    rb = (bits + 0x7FFF + lsb) & jnp.int32(-65536)
    return lax.bitcast_convert_type(rb, jnp.float32)


def _splat_lane(vec, lane):
    idx = jnp.full((16, 1), lane, jnp.int32)
    dnums = lax.GatherDimensionNumbers(
        offset_dims=(), collapsed_slice_dims=(0,), start_index_map=(0,))
    return lax.gather(vec, idx, dnums, (1,),
                      mode=lax.GatherScatterMode.PROMISE_IN_BOUNDS)


def _merge16(xk, xi, yk, yi):
    ryk = lax.rev(yk, (0,))
    ryi = lax.rev(yi, (0,))
    s = xk <= ryk
    lk = jnp.where(s, xk, ryk)
    li = jnp.where(s, xi, ryi)
    hk = jnp.where(s, ryk, xk)
    hi = jnp.where(s, ryi, xi)
    lk, li = _sort16(lk, li)
    hk, hi = _sort16(hk, hi)
    return lk, li, hk, hi


def _sc_select_combine(co2, fe2, B, V, D, P, K):
    NT = _NC * _NS
    RPT = (B * V) // NT
    TPB = NT // B
    CAP = 48
    mesh = plsc.VectorSubcoreMesh(core_axis_name="c", subcore_axis_name="s")

    @functools.partial(
        pl.kernel,
        mesh=mesh,
        out_type=jax.ShapeDtypeStruct((B * V * 2 * P,), jnp.float32),
        compiler_params=pltpu.CompilerParams(
            needs_layout_passes=False, use_tc_tiling_on_sc=False),
        scratch_types=[
            pltpu.VMEM((D * V,), jnp.float32),
            pltpu.VMEM((V,), jnp.float32),
            pltpu.VMEM((CAP,), jnp.int32),
            pltpu.VMEM((CAP,), jnp.int32),
            pltpu.VMEM((CAP,), jnp.float32),
            pltpu.VMEM((CAP,), jnp.float32),
            pltpu.VMEM((CAP, P), jnp.float32),
            pltpu.VMEM((CAP, P), jnp.float32),
            pltpu.VMEM((2 * P,), jnp.float32),
            pltpu.VMEM((2 * P,), jnp.float32),
            pltpu.SemaphoreType.DMA,
            pltpu.SemaphoreType.DMA,
        ],
    )
    def k(co_hbm, fe_hbm, out_hbm, cv, a2v, ibufA, ibufB, wbufA, wbufB,
          rowsA, rowsB, obufA, obufB, semA, semB):
        wid = lax.axis_index("s") * _NC + lax.axis_index("c")
        b = wid // TPB
        row0 = (wid % TPB) * RPT
        pltpu.sync_copy(co_hbm.at[b], cv)
        it16 = lax.iota(jnp.int32, 16)

        def prep(c, carry):
            base = c * 16
            a2 = None
            for dd in range(D):
                cvd = cv[pl.ds(dd * V + base, 16)]
                a2 = cvd * cvd if a2 is None else a2 + cvd * cvd
                cv[pl.ds(dd * V + base, 16)] = _round_bf16(cvd)
            a2v[pl.ds(base, 16)] = a2
            return carry

        lax.fori_loop(0, V // 16, prep, 0)

        def dist16(c, bc, a2r):
            base = c * 16
            cross = None
            for dd in range(D):
                cvd = cv[pl.ds(dd * V + base, 16)]
                p = cvd * bc[dd]
                cross = p if cross is None else cross + p
            d = (a2r - 2.0 * cross) + a2v[pl.ds(base, 16)]
            return d, base + it16

        def merge48(kept, dm, cidx):
            k0, k1, k2, i0, i1, i2 = kept
            cs, cis = _sort16(dm, cidx)
            rc = lax.rev(cs, (0,))
            ric = lax.rev(cis, (0,))
            s2 = k2 <= rc
            nk2 = jnp.where(s2, k2, rc)
            ni2 = jnp.where(s2, i2, ric)
            t, ti = _sort16(nk2, ni2)
            lok, loi, hik, hii = _merge16(k1, i1, t, ti)
            f0k, f0i, midk, midi = _merge16(k0, i0, lok, loi)
            f1k, f1i, f2k, f2i = _merge16(midk, midi, hik, hii)
            return f0k, f1k, f2k, f0i, f1i, f2i

        def kept_init():
            return (jnp.full((16,), _BIG), jnp.full((16,), _BIG),
                    jnp.full((16,), _BIG),
                    jnp.zeros((16,), jnp.int32), jnp.zeros((16,), jnp.int32),
                    jnp.zeros((16,), jnp.int32))

        def exact_select(bc, a2r):
            init = kept_init() + (jnp.full((16,), _BIG),)

            def chunk(c, carry):
                kept, th = carry[:6], carry[6]
                d, cidx = dist16(c, bc, a2r)
                dm = jnp.where(d < th, d, _BIG)
                kept = merge48(kept, dm, cidx)
                nth = _splat_lane(kept[2], 15)
                return kept + (nth,)

            out = lax.fori_loop(0, V // 16, chunk, init)
            return out[:6]

        _C = 6
        _NL = 2

        def select(r, ibuf, wbuf, rows, sem):
            rbase = (r // 16) * 16
            rlane = r - rbase
            bc = [_splat_lane(cv[pl.ds(dd * V + rbase, 16)], rlane)
                  for dd in range(D)]
            a2r = _splat_lane(a2v[pl.ds(rbase, 16)], rlane)

            init = tuple(jnp.full((16,), _BIG) for _ in range(_NL * _C)) + \
                tuple(jnp.zeros((16,), jnp.int32) for _ in range(_NL * _C))

            def chunk(g, carry):
                Ld = [list(carry[q * _C:(q + 1) * _C]) for q in range(_NL)]
                Li = [list(carry[(_NL + q) * _C:(_NL + q + 1) * _C])
                      for q in range(_NL)]
                for u in range(2):
                    for q in range(_NL):
                        cd, ci = dist16(2 * _NL * g + u * _NL + q, bc, a2r)
                        for i in range(_C):
                            sel = Ld[q][i] <= cd
                            nd = jnp.where(sel, Ld[q][i], cd)
                            ni = jnp.where(sel, Li[q][i], ci)
                            cd = jnp.where(sel, cd, Ld[q][i])
                            ci = jnp.where(sel, ci, Li[q][i])
                            Ld[q][i] = nd
                            Li[q][i] = ni
                return (tuple(Ld[0]) + tuple(Ld[1])
                        + tuple(Li[0]) + tuple(Li[1]))

            out = lax.fori_loop(0, V // 16 // _NL // 2, chunk, init)
            Ld = out[:_NL * _C]
            Li = out[_NL * _C:]

            kept = kept_init()
            for i in range(_NL * _C):
                kept = merge48(kept, Ld[i], Li[i])

            th40 = _splat_lane(kept[2], K - 33)
            miss = jnp.any((Ld[_C - 1] < th40) | (Ld[_NL * _C - 1] < th40))
            k0, k1, k2, i0, i1, i2 = lax.cond(
                miss, lambda _: exact_select(bc, a2r), lambda _: kept, 0)
            gb = b * V
            ibuf[pl.ds(0, 16)] = i0 + gb
            ibuf[pl.ds(16, 16)] = i1 + gb
            ibuf[pl.ds(32, 16)] = i2 + gb
            wbuf[pl.ds(0, 16)] = jnp.exp(jnp.abs(k0 * 10.0) * -1.0)
            wbuf[pl.ds(16, 16)] = jnp.exp(jnp.abs(k1 * 10.0) * -1.0)
            wbuf[pl.ds(32, 16)] = jnp.exp(jnp.abs(k2 * 10.0) * -1.0)
            return pltpu.async_copy(fe_hbm.at[ibuf], rows, sem)

        def combine(r, wbuf, rows, obuf):
            z = jnp.zeros((16,), jnp.float32)
            neg = jnp.full((16,), -_BIG)

            def cj(j, carry):
                mx0, mx1, sm0, sm1 = carry
                jbase = (j // 16) * 16
                wv = _splat_lane(wbuf[pl.ds(jbase, 16)], j - jbase)
                f0 = rows[j, pl.ds(0, 16)]
                f1 = rows[j, pl.ds(16, 16)]
                a0 = f0 * wv
                a1 = f1 * wv
                return (jnp.maximum(mx0, a0), jnp.maximum(mx1, a1),
                        sm0 + a0, sm1 + a1)

            mx0, mx1, sm0, sm1 = lax.fori_loop(1, K, cj, (neg, neg, z, z))
            inv = jnp.float32(1.0 / (K - 1))
            obuf[pl.ds(0, 16)] = mx0
            obuf[pl.ds(16, 16)] = mx1
            obuf[pl.ds(32, 16)] = sm0 * inv
            obuf[pl.ds(48, 16)] = sm1 * inv
            pltpu.sync_copy(
                obuf, out_hbm.at[pl.ds((b * V + r) * 2 * P, 2 * P)])

        def pair(g, carry):
            rA = row0 + 2 * g
            cpA = select(rA, ibufA, wbufA, rowsA, semA)
            cpB = select(rA + 1, ibufB, wbufB, rowsB, semB)
            cpA.wait()
            combine(rA, wbufA, rowsA, obufA)
            cpB.wait()
            combine(rA + 1, wbufB, rowsB, obufB)
            return carry

        lax.fori_loop(0, RPT // 2, pair, 0)

    return k(co2, fe2)


def kernel(x, Ws, bs, Wf, bf, Wo, bo):
    B, V, F = x.shape
    D = Ws.shape[1]
    P = Wf.shape[1]
    O = Wo.shape[1]
    K = 40

    coords, feats = pl.pallas_call(
        _proj_body,
        grid=(B,),
        in_specs=[
            pl.BlockSpec((1, V, F), lambda i: (i, 0, 0)),
            pl.BlockSpec((F, D), lambda i: (0, 0)),
            pl.BlockSpec((1, D), lambda i: (0, 0)),
            pl.BlockSpec((F, P), lambda i: (0, 0)),
            pl.BlockSpec((1, P), lambda i: (0, 0)),
        ],
        out_specs=[
            pl.BlockSpec((1, V, D), lambda i: (i, 0, 0)),
            pl.BlockSpec((1, V, P), lambda i: (i, 0, 0)),
        ],
        out_shape=[
            jax.ShapeDtypeStruct((B, V, D), jnp.float32),
            jax.ShapeDtypeStruct((B, V, P), jnp.float32),
        ],
    )(x, Ws, bs.reshape(1, D), Wf, bf.reshape(1, P))

    co2 = coords.transpose(0, 2, 1).reshape(B, D * V)
    fe2 = feats.reshape(B * V, P)
    coll = _sc_select_combine(co2, fe2, B, V, D, P, K)

    R = 512
    out = pl.pallas_call(
        functools.partial(_final_body, F=F),
        grid=((B * V) // R,),
        in_specs=[
            pl.BlockSpec((R, F), lambda i: (i, 0)),
            pl.BlockSpec((R, 2 * P), lambda i: (i, 0)),
            pl.BlockSpec((F + 2 * P, O), lambda i: (0, 0)),
            pl.BlockSpec((1, O), lambda i: (0, 0)),
        ],
        out_specs=pl.BlockSpec((R, O), lambda i: (i, 0)),
        out_shape=jax.ShapeDtypeStruct((B * V, O), jnp.float32),
    )(x.reshape(B * V, F), coll.reshape(B * V, 2 * P), Wo, bo.reshape(1, O))
    return out.reshape(B, V, O)


# speedup vs baseline: 1.1665x; 1.1665x over previous
import functools

import jax
import jax.numpy as jnp
from jax import lax
from jax.experimental import pallas as pl
from jax.experimental.pallas import tpu as pltpu
from jax.experimental.pallas import tpu_sc as plsc

_BIG = 3.0e38
_NC = 2
_NS = 16


def _proj_body(x_ref, Ws_ref, bs_ref, Wf_ref, bf_ref, co_ref, fe_ref):
    xa = x_ref[0]
    co_ref[0] = jnp.dot(xa, Ws_ref[...],
                        preferred_element_type=jnp.float32) + bs_ref[...]
    fe_ref[0] = jnp.dot(xa, Wf_ref[...],
                        preferred_element_type=jnp.float32) + bf_ref[...]


def _final_body(x_ref, col_ref, Wo_ref, bo_ref, out_ref, *, F):
    Wo = Wo_ref[...]
    acc = jnp.dot(x_ref[...], Wo[:F], preferred_element_type=jnp.float32)
    acc += jnp.dot(col_ref[...], Wo[F:], preferred_element_type=jnp.float32)
    out_ref[...] = jnp.tanh(acc + bo_ref[...])


def _sort16(k, v):
    return plsc.sort_key_val(k, v)


def _round_bf16(x):
    bits = lax.bitcast_convert_type(x, jnp.int32)
    lsb = lax.shift_right_logical(bits, 16) & 1
    rb = (bits + 0x7FFF + lsb) & jnp.int32(-65536)
    return lax.bitcast_convert_type(rb, jnp.float32)


def _splat_lane(vec, lane):
    idx = jnp.full((16, 1), lane, jnp.int32)
    dnums = lax.GatherDimensionNumbers(
        offset_dims=(), collapsed_slice_dims=(0,), start_index_map=(0,))
    return lax.gather(vec, idx, dnums, (1,),
                      mode=lax.GatherScatterMode.PROMISE_IN_BOUNDS)


def _merge16(xk, xi, yk, yi):
    ryk = lax.rev(yk, (0,))
    ryi = lax.rev(yi, (0,))
    s = xk <= ryk
    lk = jnp.where(s, xk, ryk)
    li = jnp.where(s, xi, ryi)
    hk = jnp.where(s, ryk, xk)
    hi = jnp.where(s, ryi, xi)
    lk, li = _sort16(lk, li)
    hk, hi = _sort16(hk, hi)
    return lk, li, hk, hi


def _sc_select_combine(co2, fe2, B, V, D, P, K):
    NT = _NC * _NS
    RPT = (B * V) // NT
    TPB = NT // B
    CAP = 48
    mesh = plsc.VectorSubcoreMesh(core_axis_name="c", subcore_axis_name="s")

    @functools.partial(
        pl.kernel,
        mesh=mesh,
        out_type=jax.ShapeDtypeStruct((B * V * 2 * P,), jnp.float32),
        compiler_params=pltpu.CompilerParams(
            needs_layout_passes=False, use_tc_tiling_on_sc=False),
        scratch_types=[
            pltpu.VMEM((D * V,), jnp.float32),
            pltpu.VMEM((V,), jnp.float32),
            pltpu.VMEM((CAP,), jnp.int32),
            pltpu.VMEM((CAP,), jnp.int32),
            pltpu.VMEM((CAP,), jnp.float32),
            pltpu.VMEM((CAP,), jnp.float32),
            pltpu.VMEM((CAP, P), jnp.float32),
            pltpu.VMEM((CAP, P), jnp.float32),
            pltpu.VMEM((2 * P,), jnp.float32),
            pltpu.VMEM((2 * P,), jnp.float32),
            pltpu.SemaphoreType.DMA,
            pltpu.SemaphoreType.DMA,
        ],
    )
    def k(co_hbm, fe_hbm, out_hbm, cv, a2v, ibufA, ibufB, wbufA, wbufB,
          rowsA, rowsB, obufA, obufB, semA, semB):
        wid = lax.axis_index("s") * _NC + lax.axis_index("c")
        b = wid // TPB
        row0 = (wid % TPB) * RPT
        pltpu.sync_copy(co_hbm.at[b], cv)
        it16 = lax.iota(jnp.int32, 16)

        def prep(c, carry):
            base = c * 16
            a2 = None
            for dd in range(D):
                cvd = cv[pl.ds(dd * V + base, 16)]
                a2 = cvd * cvd if a2 is None else a2 + cvd * cvd
                cv[pl.ds(dd * V + base, 16)] = _round_bf16(cvd)
            a2v[pl.ds(base, 16)] = a2
            return carry

        lax.fori_loop(0, V // 16, prep, 0)

        def dist16(c, bc, a2r):
            base = c * 16
            cross = None
            for dd in range(D):
                cvd = cv[pl.ds(dd * V + base, 16)]
                p = cvd * bc[dd]
                cross = p if cross is None else cross + p
            d = (a2r - 2.0 * cross) + a2v[pl.ds(base, 16)]
            return d, base + it16

        def merge48(kept, dm, cidx):
            k0, k1, k2, i0, i1, i2 = kept
            cs, cis = _sort16(dm, cidx)
            rc = lax.rev(cs, (0,))
            ric = lax.rev(cis, (0,))
            s2 = k2 <= rc
            nk2 = jnp.where(s2, k2, rc)
            ni2 = jnp.where(s2, i2, ric)
            t, ti = _sort16(nk2, ni2)
            lok, loi, hik, hii = _merge16(k1, i1, t, ti)
            f0k, f0i, midk, midi = _merge16(k0, i0, lok, loi)
            f1k, f1i, f2k, f2i = _merge16(midk, midi, hik, hii)
            return f0k, f1k, f2k, f0i, f1i, f2i

        def kept_init():
            return (jnp.full((16,), _BIG), jnp.full((16,), _BIG),
                    jnp.full((16,), _BIG),
                    jnp.zeros((16,), jnp.int32), jnp.zeros((16,), jnp.int32),
                    jnp.zeros((16,), jnp.int32))

        def exact_select(bc, a2r):
            init = kept_init() + (jnp.full((16,), _BIG),)

            def chunk(c, carry):
                kept, th = carry[:6], carry[6]
                d, cidx = dist16(c, bc, a2r)
                dm = jnp.where(d < th, d, _BIG)
                kept = merge48(kept, dm, cidx)
                nth = _splat_lane(kept[2], 15)
                return kept + (nth,)

            out = lax.fori_loop(0, V // 16, chunk, init)
            return out[:6]

        def mid_select(bc, a2r):
            C2 = 12
            init = (tuple(jnp.full((16,), _BIG) for _ in range(C2))
                    + tuple(jnp.zeros((16,), jnp.int32) for _ in range(C2)))

            def chunk(c, carry):
                Ld = list(carry[:C2])
                Li = list(carry[C2:])
                cd, ci = dist16(c, bc, a2r)
                for i in range(C2):
                    sel = Ld[i] <= cd
                    nd = jnp.where(sel, Ld[i], cd)
                    ni = jnp.where(sel, Li[i], ci)
                    cd = jnp.where(sel, cd, Ld[i])
                    ci = jnp.where(sel, ci, Li[i])
                    Ld[i] = nd
                    Li[i] = ni
                return tuple(Ld) + tuple(Li)

            out = lax.fori_loop(0, V // 16, chunk, init)
            kept = kept_init()
            for i in range(C2):
                kept = merge48(kept, out[i], out[C2 + i])
            th40 = _splat_lane(kept[2], K - 33)
            miss = jnp.any(out[C2 - 1] < th40)
            return lax.cond(miss, lambda _: exact_select(bc, a2r),
                            lambda _: kept, 0)

        _C = 6
        _NL = 2

        def select(r, ibuf, wbuf, rows, sem):
            rbase = (r // 16) * 16
            rlane = r - rbase
            bc = [_splat_lane(cv[pl.ds(dd * V + rbase, 16)], rlane)
                  for dd in range(D)]
            a2r = _splat_lane(a2v[pl.ds(rbase, 16)], rlane)

            init = tuple(jnp.full((16,), _BIG) for _ in range(_NL * _C)) + \
                tuple(jnp.zeros((16,), jnp.int32) for _ in range(_NL * _C))

            def chunk(g, carry):
                Ld = [list(carry[q * _C:(q + 1) * _C]) for q in range(_NL)]
                Li = [list(carry[(_NL + q) * _C:(_NL + q + 1) * _C])
                      for q in range(_NL)]
                for q in range(_NL):
                    cd, ci = dist16(_NL * g + q, bc, a2r)
                    for i in range(_C):
                        sel = Ld[q][i] <= cd
                        nd = jnp.where(sel, Ld[q][i], cd)
                        ni = jnp.where(sel, Li[q][i], ci)
                        cd = jnp.where(sel, cd, Ld[q][i])
                        ci = jnp.where(sel, ci, Li[q][i])
                        Ld[q][i] = nd
                        Li[q][i] = ni
                return (tuple(Ld[0]) + tuple(Ld[1])
                        + tuple(Li[0]) + tuple(Li[1]))

            out = lax.fori_loop(0, V // 16 // _NL, chunk, init)
            Ld = out[:_NL * _C]
            Li = out[_NL * _C:]

            kept = kept_init()
            for i in range(_NL * _C):
                kept = merge48(kept, Ld[i], Li[i])

            th40 = _splat_lane(kept[2], K - 33)
            miss = jnp.any((Ld[_C - 1] < th40) | (Ld[_NL * _C - 1] < th40))
            k0, k1, k2, i0, i1, i2 = lax.cond(
                miss, lambda _: mid_select(bc, a2r), lambda _: kept, 0)
            gb = b * V
            ibuf[pl.ds(0, 16)] = i0 + gb
            ibuf[pl.ds(16, 16)] = i1 + gb
            ibuf[pl.ds(32, 16)] = i2 + gb
            wbuf[pl.ds(0, 16)] = jnp.exp(jnp.abs(k0 * 10.0) * -1.0)
            wbuf[pl.ds(16, 16)] = jnp.exp(jnp.abs(k1 * 10.0) * -1.0)
            wbuf[pl.ds(32, 16)] = jnp.exp(jnp.abs(k2 * 10.0) * -1.0)
            return pltpu.async_copy(fe_hbm.at[ibuf], rows, sem)

        def combine(r, wbuf, rows, obuf):
            z = jnp.zeros((16,), jnp.float32)
            neg = jnp.full((16,), -_BIG)

            def cj(j, carry):
                mx0, mx1, sm0, sm1 = carry
                jbase = (j // 16) * 16
                wv = _splat_lane(wbuf[pl.ds(jbase, 16)], j - jbase)
                f0 = rows[j, pl.ds(0, 16)]
                f1 = rows[j, pl.ds(16, 16)]
                a0 = f0 * wv
                a1 = f1 * wv
                return (jnp.maximum(mx0, a0), jnp.maximum(mx1, a1),
                        sm0 + a0, sm1 + a1)

            mx0, mx1, sm0, sm1 = lax.fori_loop(1, K, cj, (neg, neg, z, z))
            inv = jnp.float32(1.0 / (K - 1))
            obuf[pl.ds(0, 16)] = mx0
            obuf[pl.ds(16, 16)] = mx1
            obuf[pl.ds(32, 16)] = sm0 * inv
            obuf[pl.ds(48, 16)] = sm1 * inv
            pltpu.sync_copy(
                obuf, out_hbm.at[pl.ds((b * V + r) * 2 * P, 2 * P)])

        def pair(g, carry):
            rA = row0 + 2 * g
            cpA = select(rA, ibufA, wbufA, rowsA, semA)
            cpB = select(rA + 1, ibufB, wbufB, rowsB, semB)
            cpA.wait()
            combine(rA, wbufA, rowsA, obufA)
            cpB.wait()
            combine(rA + 1, wbufB, rowsB, obufB)
            return carry

        lax.fori_loop(0, RPT // 2, pair, 0)

    return k(co2, fe2)


def kernel(x, Ws, bs, Wf, bf, Wo, bo):
    B, V, F = x.shape
    D = Ws.shape[1]
    P = Wf.shape[1]
    O = Wo.shape[1]
    K = 40

    coords, feats = pl.pallas_call(
        _proj_body,
        grid=(B,),
        in_specs=[
            pl.BlockSpec((1, V, F), lambda i: (i, 0, 0)),
            pl.BlockSpec((F, D), lambda i: (0, 0)),
            pl.BlockSpec((1, D), lambda i: (0, 0)),
            pl.BlockSpec((F, P), lambda i: (0, 0)),
            pl.BlockSpec((1, P), lambda i: (0, 0)),
        ],
        out_specs=[
            pl.BlockSpec((1, V, D), lambda i: (i, 0, 0)),
            pl.BlockSpec((1, V, P), lambda i: (i, 0, 0)),
        ],
        out_shape=[
            jax.ShapeDtypeStruct((B, V, D), jnp.float32),
            jax.ShapeDtypeStruct((B, V, P), jnp.float32),
        ],
    )(x, Ws, bs.reshape(1, D), Wf, bf.reshape(1, P))

    co2 = coords.transpose(0, 2, 1).reshape(B, D * V)
    fe2 = feats.reshape(B * V, P)
    coll = _sc_select_combine(co2, fe2, B, V, D, P, K)

    R = 512
    out = pl.pallas_call(
        functools.partial(_final_body, F=F),
        grid=((B * V) // R,),
        in_specs=[
            pl.BlockSpec((R, F), lambda i: (i, 0)),
            pl.BlockSpec((R, 2 * P), lambda i: (i, 0)),
            pl.BlockSpec((F + 2 * P, O), lambda i: (0, 0)),
            pl.BlockSpec((1, O), lambda i: (0, 0)),
        ],
        out_specs=pl.BlockSpec((R, O), lambda i: (i, 0)),
        out_shape=jax.ShapeDtypeStruct((B * V, O), jnp.float32),
    )(x.reshape(B * V, F), coll.reshape(B * V, 2 * P), Wo, bo.reshape(1, O))
    return out.reshape(B, V, O)
